# Initial kernel scaffold; baseline (speedup 1.0000x reference)
import jax
import jax.numpy as jnp
from jax.experimental import pallas as pl


def kernel(length, relative_attention_bias):
    raise NotImplementedError("write your pallas kernel here")


# speedup vs baseline: 5.6132x; 5.6132x over previous
import functools

import jax
import jax.numpy as jnp
from jax import lax
from jax.experimental import pallas as pl
from jax.experimental.pallas import tpu as pltpu
from jax.experimental.pallas import tpu_sc as plsc

MAX_REL = 1024
L = 1024
D = 64
NC = 2
NS = 16
NW = NC * NS
SLABS_PER_W = L // NW
SPAN = 2 * MAX_REL - 8

_LO = [8, 7, 6, 5, 4, 3, 2, 1]


@functools.partial(
    pl.kernel,
    mesh=plsc.VectorSubcoreMesh(core_axis_name="c", subcore_axis_name="s"),
    out_type=jax.ShapeDtypeStruct((L, L, D), jnp.float32),
    scratch_types=[
        pltpu.VMEM_SHARED((8, SPAN, D), jnp.float32),
        pltpu.SemaphoreType.DMA,
    ],
)
def _rpe_sc(stack_hbm, out_hbm, shared, sem):
    cid = lax.axis_index("c")
    sid = lax.axis_index("s")
    wid = cid * NS + sid

    @pl.when(sid == 0)
    def _stage():
        pltpu.sync_copy(stack_hbm, shared)

    plsc.subcore_barrier()

    layer = wid % 8
    q = wid // 8
    copies = []
    for t in range(SLABS_PER_W):
        i = wid + t * NW
        off = pl.multiple_of(8 * (127 - 4 * t - q), 8)
        copies.append(
            pltpu.async_copy(
                shared.at[layer, pl.ds(off, L)],
                out_hbm.at[i],
                sem,
            )
        )
    for c in copies:
        c.wait()


def kernel(length, relative_attention_bias):
    del length
    stack = jnp.stack(
        [lax.slice_in_dim(relative_attention_bias, lo, lo + SPAN) for lo in _LO]
    )
    return _rpe_sc(stack)


# speedup vs baseline: 5.8736x; 1.0464x over previous
import functools

import jax
import jax.numpy as jnp
from jax import lax
from jax.experimental import pallas as pl
from jax.experimental.pallas import tpu as pltpu
from jax.experimental.pallas import tpu_sc as plsc

MAX_REL = 1024
L = 1024
D = 64
NC = 2
NS = 16
NW = NC * NS
SLABS_PER_W = L // NW
NSHIFT = 16
PAIR_ROWS = 1016


@functools.partial(
    pl.kernel,
    mesh=plsc.VectorSubcoreMesh(core_axis_name="c", subcore_axis_name="s"),
    out_type=jax.ShapeDtypeStruct((L, L // 2, 2 * D), jnp.float32),
    scratch_types=[
        pltpu.VMEM_SHARED((NSHIFT, PAIR_ROWS, 2 * D), jnp.float32),
        pltpu.SemaphoreType.DMA,
    ],
)
def _rpe_sc(stack_hbm, out_hbm, shared, sem):
    cid = lax.axis_index("c")
    sid = lax.axis_index("s")
    wid = cid * NS + sid

    @pl.when(sid == 0)
    def _stage():
        pltpu.sync_copy(stack_hbm, shared)

    plsc.subcore_barrier()

    layer = wid % NSHIFT
    q = wid // NSHIFT
    copies = []
    for t in range(SLABS_PER_W):
        i = wid + t * NW
        m0 = pl.multiple_of(8 * (63 - 2 * t - q), 8)
        copies.append(
            pltpu.async_copy(
                shared.at[layer, pl.ds(m0, L // 2)],
                out_hbm.at[i],
                sem,
            )
        )
    for c in copies:
        c.wait()


def kernel(length, relative_attention_bias):
    del length
    stack = jnp.stack(
        [
            relative_attention_bias[NSHIFT - u : 2 * MAX_REL - u].reshape(
                PAIR_ROWS, 2 * D
            )
            for u in range(NSHIFT)
        ]
    )
    packed = _rpe_sc(stack)
    return packed.reshape(L, L, D)


# speedup vs baseline: 11.8948x; 2.0251x over previous
import functools

import jax
import jax.numpy as jnp
from jax import lax
from jax.experimental import pallas as pl
from jax.experimental.pallas import tpu as pltpu
from jax.experimental.pallas import tpu_sc as plsc

MAX_REL = 1024
L = 1024
D = 64
NC = 2
NS = 16
NW = NC * NS
NGROUPS = 4
SLABS_PER_G = 8
MM = 15


@functools.partial(
    pl.kernel,
    mesh=plsc.VectorSubcoreMesh(core_axis_name="c", subcore_axis_name="s"),
    out_type=jax.ShapeDtypeStruct((L, 8, 8, 8, 128), jnp.float32),
    scratch_types=[
        pltpu.VMEM((8, MM, 8, 128), jnp.float32),
        pltpu.SemaphoreType.DMA,
    ],
)
def _rpe_sc(stack_hbm, out_hbm, layer_buf, sem):
    cid = lax.axis_index("c")
    sid = lax.axis_index("s")
    wid = cid * NS + sid

    for g in range(NGROUPS):
        layer = lax.rem(2 * 128 - wid - 32 * g, 128)
        pltpu.sync_copy(stack_hbm.at[layer], layer_buf)
        copies = []
        for tt in range(SLABS_PER_G):
            i = wid + 32 * g + 128 * tt
            mm0 = 7 - tt
            for dt in range(8):
                copies.append(
                    pltpu.async_copy(
                        layer_buf.at[dt, pl.ds(mm0, 8)],
                        out_hbm.at[i, dt],
                        sem,
                    )
                )
        for c in copies:
            c.wait()


def kernel(length, relative_attention_bias):
    del length
    layers = []
    for r in range(128):
        c = ((r - 1) % 128) + 1
        raw = relative_attention_bias[c : c + 128 * MM].reshape(MM, 128, 8, 8)
        layers.append(raw.transpose(2, 0, 3, 1))
    stack = jnp.stack(layers)
    k2 = _rpe_sc(stack)
    return k2.transpose(0, 2, 4, 1, 3).reshape(L, L, D)


# speedup vs baseline: 18.8346x; 1.5834x over previous
import functools

import jax
import jax.numpy as jnp
from jax import lax
from jax.experimental import pallas as pl
from jax.experimental.pallas import tpu as pltpu
from jax.experimental.pallas import tpu_sc as plsc

MAX_REL = 1024
L = 1024
D = 64
NC = 2
NS = 16
NW = NC * NS
NGROUPS = 4
SLABS_PER_G = 8
COLS = 15 * 128


PADC = 17 * 128


def _build_body(t_ref, o_ref):
    r = pl.program_id(0)
    c = jnp.where(r == 0, 128, r)
    rolled = pltpu.roll(t_ref[...], PADC - c, 2)
    o_ref[0] = rolled[:, :, :COLS]


_build_tc = pl.pallas_call(
    _build_body,
    grid=(128,),
    in_specs=[pl.BlockSpec((8, 8, PADC), lambda r: (0, 0, 0))],
    out_specs=pl.BlockSpec((1, 8, 8, COLS), lambda r: (r, 0, 0, 0)),
    out_shape=jax.ShapeDtypeStruct((128, 8, 8, COLS), jnp.float32),
    compiler_params=pltpu.CompilerParams(
        dimension_semantics=("arbitrary",),
    ),
)


@functools.partial(
    pl.kernel,
    mesh=plsc.VectorSubcoreMesh(core_axis_name="c", subcore_axis_name="s"),
    out_type=jax.ShapeDtypeStruct((L, 8, 8, 8, 128), jnp.float32),
    scratch_types=[
        pltpu.VMEM((8, 8, COLS), jnp.float32),
        pltpu.SemaphoreType.DMA,
    ],
)
def _rpe_sc(stack_hbm, out_hbm, layer_buf, sem):
    cid = lax.axis_index("c")
    sid = lax.axis_index("s")
    wid = cid * NS + sid

    for g in range(NGROUPS):
        layer = lax.rem(2 * 128 - wid - 32 * g, 128)
        pltpu.sync_copy(stack_hbm.at[layer], layer_buf)
        copies = []
        for tt in range(SLABS_PER_G):
            i = wid + 32 * g + 128 * tt
            mm0 = 7 - tt
            for jt in range(8):
                copies.append(
                    pltpu.async_copy(
                        layer_buf.at[:, :, pl.ds(128 * (mm0 + jt), 128)],
                        out_hbm.at[i, :, jt],
                        sem,
                    )
                )
        for c in copies:
            c.wait()


def kernel(length, relative_attention_bias):
    del length
    t3 = relative_attention_bias.T.reshape(8, 8, 2 * MAX_REL + 1)
    t3p = jnp.pad(t3, ((0, 0), (0, 0), (0, PADC - (2 * MAX_REL + 1))))
    k2 = _rpe_sc(_build_tc(t3p))
    return k2.transpose(0, 2, 4, 1, 3).reshape(L, L, D)


# speedup vs baseline: 22.7667x; 1.2088x over previous
import functools

import jax
import jax.numpy as jnp
from jax import lax
from jax.experimental import pallas as pl
from jax.experimental.pallas import tpu as pltpu
from jax.experimental.pallas import tpu_sc as plsc

MAX_REL = 1024
L = 1024
D = 64
NC = 2
NS = 16
NW = NC * NS
NGROUPS = 4
SLABS_PER_G = 8
COLS = 15 * 128


PADC = 17 * 128


LAYERS_PER_STEP = 4


def _build_body(t_ref, o_ref):
    r4 = pl.program_id(0)
    t = t_ref[...]
    for k in range(LAYERS_PER_STEP):
        r = r4 * LAYERS_PER_STEP + k
        c = jnp.where(r == 0, 128, r)
        o_ref[k] = pltpu.roll(t, PADC - c, 2)[:, :, :COLS]


_build_tc = pl.pallas_call(
    _build_body,
    grid=(128 // LAYERS_PER_STEP,),
    in_specs=[pl.BlockSpec((8, 8, PADC), lambda r: (0, 0, 0))],
    out_specs=pl.BlockSpec((LAYERS_PER_STEP, 8, 8, COLS), lambda r: (r, 0, 0, 0)),
    out_shape=jax.ShapeDtypeStruct((128, 8, 8, COLS), jnp.float32),
    compiler_params=pltpu.CompilerParams(
        dimension_semantics=("arbitrary",),
    ),
)


@functools.partial(
    pl.kernel,
    mesh=plsc.VectorSubcoreMesh(core_axis_name="c", subcore_axis_name="s"),
    out_type=jax.ShapeDtypeStruct((L, 8, 8, 8, 128), jnp.float32),
    scratch_types=[
        pltpu.VMEM((8, 8, COLS), jnp.float32),
        pltpu.SemaphoreType.DMA,
    ],
)
def _rpe_sc(stack_hbm, out_hbm, layer_buf, sem):
    cid = lax.axis_index("c")
    sid = lax.axis_index("s")
    wid = cid * NS + sid

    for g in range(NGROUPS):
        layer = lax.rem(2 * 128 - wid - 32 * g, 128)
        pltpu.sync_copy(stack_hbm.at[layer], layer_buf)
        copies = []
        for tt in range(SLABS_PER_G):
            i = wid + 32 * g + 128 * tt
            mm0 = 7 - tt
            for jt in range(8):
                copies.append(
                    pltpu.async_copy(
                        layer_buf.at[:, :, pl.ds(128 * (mm0 + jt), 128)],
                        out_hbm.at[i, :, jt],
                        sem,
                    )
                )
        for c in copies:
            c.wait()


def kernel(length, relative_attention_bias):
    del length
    t3 = relative_attention_bias.T.reshape(8, 8, 2 * MAX_REL + 1)
    t3p = jnp.pad(t3, ((0, 0), (0, 0), (0, PADC - (2 * MAX_REL + 1))))
    k2 = _rpe_sc(_build_tc(t3p))
    return k2.transpose(0, 2, 4, 1, 3).reshape(L, L, D)


# speedup vs baseline: 23.3730x; 1.0266x over previous
import functools

import jax
import jax.numpy as jnp
from jax import lax
from jax.experimental import pallas as pl
from jax.experimental.pallas import tpu as pltpu
from jax.experimental.pallas import tpu_sc as plsc

MAX_REL = 1024
L = 1024
D = 64
NC = 2
NS = 16
NW = NC * NS
NGROUPS = 4
SLABS_PER_G = 8
COLS = 15 * 128


PADC = 17 * 128


LAYERS_PER_STEP = 8


def _build_body(t_ref, o_ref):
    r4 = pl.program_id(0)
    t = t_ref[...]
    for k in range(LAYERS_PER_STEP):
        r = r4 * LAYERS_PER_STEP + k
        c = jnp.where(r == 0, 128, r)
        o_ref[k] = pltpu.roll(t, PADC - c, 2)[:, :, :COLS]


_build_tc = pl.pallas_call(
    _build_body,
    grid=(128 // LAYERS_PER_STEP,),
    in_specs=[pl.BlockSpec((8, 8, PADC), lambda r: (0, 0, 0))],
    out_specs=pl.BlockSpec((LAYERS_PER_STEP, 8, 8, COLS), lambda r: (r, 0, 0, 0)),
    out_shape=jax.ShapeDtypeStruct((128, 8, 8, COLS), jnp.float32),
    compiler_params=pltpu.CompilerParams(
        dimension_semantics=("arbitrary",),
    ),
)


@functools.partial(
    pl.kernel,
    mesh=plsc.VectorSubcoreMesh(core_axis_name="c", subcore_axis_name="s"),
    out_type=jax.ShapeDtypeStruct((L, 8, 8, 8, 128), jnp.float32),
    scratch_types=[
        pltpu.VMEM((8, 8, COLS), jnp.float32),
        pltpu.SemaphoreType.DMA,
    ],
)
def _rpe_sc(stack_hbm, out_hbm, layer_buf, sem):
    cid = lax.axis_index("c")
    sid = lax.axis_index("s")
    wid = cid * NS + sid

    for g in range(NGROUPS):
        layer = lax.rem(2 * 128 - wid - 32 * g, 128)
        pltpu.sync_copy(stack_hbm.at[layer], layer_buf)
        copies = []
        for tt in range(SLABS_PER_G):
            i = wid + 32 * g + 128 * tt
            mm0 = 7 - tt
            for jt in range(8):
                copies.append(
                    pltpu.async_copy(
                        layer_buf.at[:, :, pl.ds(128 * (mm0 + jt), 128)],
                        out_hbm.at[i, :, jt],
                        sem,
                    )
                )
        for c in copies:
            c.wait()


def kernel(length, relative_attention_bias):
    del length
    t3 = relative_attention_bias.T.reshape(8, 8, 2 * MAX_REL + 1)
    t3p = jnp.pad(t3, ((0, 0), (0, 0), (0, PADC - (2 * MAX_REL + 1))))
    k2 = _rpe_sc(_build_tc(t3p))
    return k2.transpose(0, 2, 4, 1, 3).reshape(L, L, D)


# speedup vs baseline: 23.8345x; 1.0197x over previous
import functools

import jax
import jax.numpy as jnp
from jax import lax
from jax.experimental import pallas as pl
from jax.experimental.pallas import tpu as pltpu
from jax.experimental.pallas import tpu_sc as plsc

MAX_REL = 1024
L = 1024
D = 64
NC = 2
NS = 16
NW = NC * NS
NGROUPS = 4
SLABS_PER_G = 8
COLS = 15 * 128


PADC = 16 * 128


LAYERS_PER_STEP = 16


def _build_body(t_ref, o_ref):
    r4 = pl.program_id(0)
    t = t_ref[...]
    for k in range(LAYERS_PER_STEP):
        r = r4 * LAYERS_PER_STEP + k
        c = jnp.where(r == 0, 128, r)
        o_ref[k] = pltpu.roll(t, PADC - c, 2)[:, :, :COLS]


_build_tc = pl.pallas_call(
    _build_body,
    grid=(128 // LAYERS_PER_STEP,),
    in_specs=[pl.BlockSpec((8, 8, PADC), lambda r: (0, 0, 0))],
    out_specs=pl.BlockSpec((LAYERS_PER_STEP, 8, 8, COLS), lambda r: (r, 0, 0, 0)),
    out_shape=jax.ShapeDtypeStruct((128, 8, 8, COLS), jnp.float32),
    compiler_params=pltpu.CompilerParams(
        dimension_semantics=("arbitrary",),
    ),
)


@functools.partial(
    pl.kernel,
    mesh=plsc.VectorSubcoreMesh(core_axis_name="c", subcore_axis_name="s"),
    out_type=jax.ShapeDtypeStruct((L, 8, 8, 8, 128), jnp.float32),
    scratch_types=[
        pltpu.VMEM((8, 8, COLS), jnp.float32),
        pltpu.SemaphoreType.DMA,
    ],
)
def _rpe_sc(stack_hbm, out_hbm, layer_buf, sem):
    cid = lax.axis_index("c")
    sid = lax.axis_index("s")
    wid = cid * NS + sid

    for g in range(NGROUPS):
        layer = lax.rem(2 * 128 - wid - 32 * g, 128)
        pltpu.sync_copy(stack_hbm.at[layer], layer_buf)
        copies = []
        for tt in range(SLABS_PER_G):
            i = wid + 32 * g + 128 * tt
            mm0 = 7 - tt
            for jt in range(8):
                copies.append(
                    pltpu.async_copy(
                        layer_buf.at[:, :, pl.ds(128 * (mm0 + jt), 128)],
                        out_hbm.at[i, :, jt],
                        sem,
                    )
                )
        for c in copies:
            c.wait()


def kernel(length, relative_attention_bias):
    del length
    t3 = relative_attention_bias.T.reshape(8, 8, 2 * MAX_REL + 1)
    t3p = t3[:, :, :PADC]
    k2 = _rpe_sc(_build_tc(t3p))
    return k2.transpose(0, 2, 4, 1, 3).reshape(L, L, D)
